# Initial kernel scaffold; baseline (speedup 1.0000x reference)
import jax
import jax.numpy as jnp
from jax.experimental import pallas as pl


def kernel(traj, trans_matrix):
    raise NotImplementedError("write your pallas kernel here")


# speedup vs baseline: 1.3392x; 1.3392x over previous
import functools

import jax
import jax.numpy as jnp
from jax import lax
from jax.experimental import pallas as pl
from jax.experimental.pallas import tpu as pltpu
from jax.experimental.pallas import tpu_sc as plsc

_L = 10000
_B = 4096
_NC = 2
_NS = 16
_NW = _NC * _NS
_BPW = _B // _NW
_R = 8
_NG = _BPW // _R


def _sc_lookup(last_loc, trans_matrix):
    mesh = plsc.VectorSubcoreMesh(core_axis_name="c", subcore_axis_name="s")

    @functools.partial(
        pl.kernel,
        mesh=mesh,
        out_type=jax.ShapeDtypeStruct((_B, _L), jnp.float32),
        scratch_types=[
            pltpu.VMEM((_BPW + 16,), jnp.int32),
            *[pltpu.VMEM((1, _L), jnp.float32) for _ in range(_R)],
            *[pltpu.SemaphoreType.DMA for _ in range(_R)],
        ],
    )
    def body(idx_hbm, table_hbm, out_hbm, idx_v, *rest):
        bufs, sems = rest[:_R], rest[_R:]
        wid = lax.axis_index("s") * _NC + lax.axis_index("c")
        base = wid * _BPW

        pltpu.sync_copy(idx_hbm.at[pl.ds(base, _BPW)],
                        idx_v.at[pl.ds(0, _BPW)])

        def start_gather(row, s):
            pltpu.make_async_copy(
                table_hbm.at[pl.ds(row, 1)], bufs[s], sems[s]).start()

        def wait_gather(s):
            pltpu.make_async_copy(
                table_hbm.at[pl.ds(0, 1)], bufs[s], sems[s]).wait()

        v0 = idx_v[pl.ds(0, 16)]
        for s in range(_R):
            start_gather(v0[s], s)

        def step(q, carry):
            off = pl.multiple_of(q * 16, 8)
            vq = idx_v[pl.ds(off, 16)]
            for s in range(_R):
                wait_gather(s)
                pltpu.sync_copy(
                    bufs[s], out_hbm.at[pl.ds(base + q * 16 + s, 1)])
                start_gather(vq[8 + s], s)
            offn = pl.multiple_of(q * 16 + 16, 8)
            vn = idx_v[pl.ds(offn, 16)]
            for s in range(_R):
                wait_gather(s)
                pltpu.sync_copy(
                    bufs[s], out_hbm.at[pl.ds(base + q * 16 + 8 + s, 1)])

                @pl.when(q * 16 + 16 + s < _BPW)
                def _(s=s, vn=vn):
                    start_gather(vn[s], s)

            return carry

        lax.fori_loop(0, _NG // 2, step, 0)

    return body(last_loc, trans_matrix)


def kernel(traj, trans_matrix):
    last_loc = traj[:, -1, 1].astype(jnp.int32)
    return _sc_lookup(last_loc, trans_matrix)
